# baseline (speedup 1.0000x reference)
import jax
import jax.numpy as jnp
from jax import lax
from jax.experimental import pallas as pl
from jax.experimental.pallas import tpu as pltpu
from jax.experimental.pallas import tpu_sc as plsc

B = 16384
L = 20
D = 64
DP = 72
ZCH = 1000000
HASH_MUL = -1640531535

NC, NS = 2, 16
NW = NC * NS
NI = B * L // NW
CH = 80
NCH = NI // CH
BPC = CH // L
NB = B // NW
NBUF = 4
VL = 16


def _body(idxt_hbm, table_hbm, out_hbm, idxv, hidx, rows, outb,
          s0, s1, s2, s3):
    sems = (s0, s1, s2, s3)
    cid = lax.axis_index("c")
    sid = lax.axis_index("s")
    wid = sid * NC + cid
    base_b = wid * NB

    pltpu.sync_copy(idxt_hbm.at[:, pl.ds(base_b, NB)], idxv)

    iot = lax.iota(jnp.int32, VL)

    def hash_col(j, carry):
        lb = j * VL + iot
        row = lax.shift_right_logical(lb, jnp.int32(2))
        colb = (lb & 3) * L
        for l in range(L):
            x = idxv[jnp.int32(l), pl.ds(j * VL, VL)]
            t = x * jnp.int32(HASH_MUL)
            q = (t.astype(jnp.float32) * jnp.float32(1e-6)).astype(jnp.int32)
            r = t - q * jnp.int32(ZCH)
            r = jnp.where(r < 0, r + jnp.int32(ZCH), r)
            r = jnp.where(r >= jnp.int32(ZCH), r - jnp.int32(ZCH), r)
            plsc.store_scatter(hidx, [row, colb + l], r)
        return carry

    lax.fori_loop(jnp.int32(0), jnp.int32(NB // VL), hash_col, jnp.int32(0))

    for b in range(NBUF):
        b32 = jnp.int32(b)
        pltpu.make_async_copy(
            table_hbm.at[hidx.at[b32]], rows.at[b32], sems[b]).start()

    def group(gi, carry):
        g = gi * NBUF
        for b in range(NBUF):
            b32 = jnp.int32(b)
            c = g + b
            pltpu.make_async_copy(
                table_hbm.at[hidx.at[c]], rows.at[b32], sems[b]).wait()
            for bb in range(BPC):
                row0 = bb * L
                accs = [rows[b32, jnp.int32(row0), pl.ds(j * VL, VL)]
                        for j in range(D // VL)]
                for l in range(1, L):
                    for j in range(D // VL):
                        accs[j] = accs[j] + rows[
                            b32, jnp.int32(row0 + l), pl.ds(j * VL, VL)]
                ob = c * BPC + bb
                for j in range(D // VL):
                    outb[ob, pl.ds(j * VL, VL)] = accs[j]
            nc_ = c + NBUF

            @pl.when(nc_ < NCH)
            def _start_next():
                pltpu.make_async_copy(
                    table_hbm.at[hidx.at[nc_]], rows.at[b32], sems[b]).start()
        return carry

    lax.fori_loop(jnp.int32(0), jnp.int32(NCH // NBUF), group, jnp.int32(0))

    pltpu.sync_copy(outb, out_hbm.at[pl.ds(wid * NB, NB)])


def kernel(indices, table):
    idxt = indices.astype(jnp.int32).T
    tpad = jnp.pad(table, ((0, 0), (0, DP - D)))
    run = pl.kernel(
        _body,
        out_type=jax.ShapeDtypeStruct((B, D), jnp.float32),
        mesh=plsc.VectorSubcoreMesh(
            core_axis_name="c", subcore_axis_name="s",
            num_cores=NC, num_subcores=NS),
        scratch_types=[
            pltpu.VMEM((L, NB), jnp.int32),
            pltpu.VMEM((NCH, CH), jnp.int32),
            pltpu.VMEM((NBUF, CH, DP), jnp.float32),
            pltpu.VMEM((NB, D), jnp.float32),
            pltpu.SemaphoreType.DMA,
            pltpu.SemaphoreType.DMA,
            pltpu.SemaphoreType.DMA,
            pltpu.SemaphoreType.DMA,
        ],
        compiler_params=pltpu.CompilerParams(
            use_tc_tiling_on_sc=False, needs_layout_passes=False),
    )
    return run(idxt, tpad)


# speedup vs baseline: 1.7990x; 1.7990x over previous
import jax
import jax.numpy as jnp
from jax import lax
from jax.experimental import pallas as pl
from jax.experimental.pallas import tpu as pltpu
from jax.experimental.pallas import tpu_sc as plsc

B = 16384
L = 20
D = 64
DP = 128
ZCH = 1000000
HASH_MUL = -1640531535

NC, NS = 2, 16
NW = NC * NS
NI = B * L // NW
CH = 80
NCH = NI // CH
BPC = CH // L
NB = B // NW
NBUF = 4
VL = 16


def _body(idxt_hbm, table_hbm, out_hbm, idxv, hidx, rows, outb,
          s0, s1, s2, s3):
    sems = (s0, s1, s2, s3)
    cid = lax.axis_index("c")
    sid = lax.axis_index("s")
    wid = sid * NC + cid
    base_b = wid * NB

    pltpu.sync_copy(idxt_hbm.at[:, pl.ds(base_b, NB)], idxv)

    iot = lax.iota(jnp.int32, VL)

    def hash_col(j, carry):
        lb = j * VL + iot
        row = lax.shift_right_logical(lb, jnp.int32(2))
        colb = (lb & 3) * L
        for l in range(L):
            x = idxv[jnp.int32(l), pl.ds(j * VL, VL)]
            t = x * jnp.int32(HASH_MUL)
            q = (t.astype(jnp.float32) * jnp.float32(1e-6)).astype(jnp.int32)
            r = t - q * jnp.int32(ZCH)
            r = jnp.where(r < 0, r + jnp.int32(ZCH), r)
            r = jnp.where(r >= jnp.int32(ZCH), r - jnp.int32(ZCH), r)
            plsc.store_scatter(hidx, [row, colb + l], r)
        return carry

    lax.fori_loop(jnp.int32(0), jnp.int32(NB // VL), hash_col, jnp.int32(0))

    for b in range(NBUF):
        b32 = jnp.int32(b)
        pltpu.make_async_copy(
            table_hbm.at[hidx.at[b32]], rows.at[b32], sems[b]).start()

    def group(gi, carry):
        g = gi * NBUF
        for b in range(NBUF):
            b32 = jnp.int32(b)
            c = g + b
            pltpu.make_async_copy(
                table_hbm.at[hidx.at[c]], rows.at[b32], sems[b]).wait()
            for bb in range(BPC):
                row0 = bb * L
                accs = [rows[b32, jnp.int32(row0), pl.ds(j * VL, VL)]
                        for j in range(D // VL)]
                for l in range(1, L):
                    for j in range(D // VL):
                        accs[j] = accs[j] + rows[
                            b32, jnp.int32(row0 + l), pl.ds(j * VL, VL)]
                ob = c * BPC + bb
                for j in range(D // VL):
                    outb[ob, pl.ds(j * VL, VL)] = accs[j]
            nc_ = c + NBUF

            @pl.when(nc_ < NCH)
            def _start_next():
                pltpu.make_async_copy(
                    table_hbm.at[hidx.at[nc_]], rows.at[b32], sems[b]).start()
        return carry

    lax.fori_loop(jnp.int32(0), jnp.int32(NCH // NBUF), group, jnp.int32(0))

    pltpu.sync_copy(outb, out_hbm.at[pl.ds(wid * NB, NB)])


def kernel(indices, table):
    idxt = indices.astype(jnp.int32).T
    tpad = jnp.pad(table, ((0, 0), (0, DP - D)))
    run = pl.kernel(
        _body,
        out_type=jax.ShapeDtypeStruct((B, D), jnp.float32),
        mesh=plsc.VectorSubcoreMesh(
            core_axis_name="c", subcore_axis_name="s",
            num_cores=NC, num_subcores=NS),
        scratch_types=[
            pltpu.VMEM((L, NB), jnp.int32),
            pltpu.VMEM((NCH, CH), jnp.int32),
            pltpu.VMEM((NBUF, CH, DP), jnp.float32),
            pltpu.VMEM((NB, D), jnp.float32),
            pltpu.SemaphoreType.DMA,
            pltpu.SemaphoreType.DMA,
            pltpu.SemaphoreType.DMA,
            pltpu.SemaphoreType.DMA,
        ],
        compiler_params=pltpu.CompilerParams(
            use_tc_tiling_on_sc=False, needs_layout_passes=False),
    )
    return run(idxt, tpad)
